# baseline (speedup 1.0000x reference)
import functools

import jax
import jax.numpy as jnp
from jax import lax
from jax.experimental import pallas as pl
from jax.experimental.pallas import tpu as pltpu, tpu_sc as plsc


def _make_sc_gather(V, D, B):
    info = plsc.get_sparse_core_info()
    L = info.num_lanes
    NW = info.num_cores * info.num_subcores
    assert B % (8 * NW) == 0 and D % L == 0
    b_per_w = B // NW
    n_groups = b_per_w // L
    mesh = plsc.VectorSubcoreMesh(core_axis_name="c", subcore_axis_name="s")

    @functools.partial(
        pl.kernel,
        mesh=mesh,
        out_type=jax.ShapeDtypeStruct((B, 2 * D), jnp.float32),
        scratch_types=[
            pltpu.VMEM((b_per_w,), jnp.int32),
            pltpu.VMEM((b_per_w,), jnp.int32),
            pltpu.VMEM((b_per_w, 2 * D), jnp.float32),
            pltpu.SemaphoreType.DMA,
        ],
    )
    def emb(labels_hbm, table_hbm, out_hbm, lab_v, blk_v, pairs_v, sem):
        wid = lax.axis_index("s") * info.num_cores + lax.axis_index("c")
        base = wid * b_per_w
        pltpu.sync_copy(labels_hbm.at[pl.ds(base, b_per_w)], lab_v)

        def blk_body(g, _):
            vec = lab_v[pl.ds(g * L, L)]
            blk_v[pl.ds(g * L, L)] = lax.shift_right_logical(vec, 1)
            return 0

        lax.fori_loop(0, n_groups, blk_body, 0)
        pltpu.async_copy(table_hbm.at[blk_v], pairs_v, sem).wait()

        def sel_body(g, _):
            vec = lab_v[pl.ds(g * L, L)]
            for j in range(L):
                i = g * L + j

                @pl.when(lax.rem(vec[j], 2) != 0)
                def _():
                    for q in range(D // L):
                        pairs_v[i, pl.ds(q * L, L)] = pairs_v[
                            i, pl.ds(D + q * L, L)
                        ]

            return 0

        lax.fori_loop(0, n_groups, sel_body, 0)
        pltpu.sync_copy(pairs_v, out_hbm.at[pl.ds(base, b_per_w)])

    return emb


def kernel(labels, embedding_table):
    B = labels.shape[0]
    V, D = embedding_table.shape
    emb = _make_sc_gather(V, D, B)
    table2 = embedding_table.reshape(V // 2, 2 * D)
    out2 = emb(labels.astype(jnp.int32), table2)
    return out2[:, :D]
